# baseline (speedup 1.0000x reference)
import functools

import numpy as np
import jax
import jax.numpy as jnp
from jax import lax
from jax.experimental import pallas as pl
from jax.experimental.pallas import tpu as pltpu
from jax.experimental.pallas import tpu_sc as plsc

L = 16
NC, NS = 2, 16
NW = NC * NS


def _positional_encoding(seq_len, d_model):
    pos = np.arange(seq_len)[:, np.newaxis]
    dim = np.arange(d_model)[np.newaxis, :]
    angles = pos / np.power(10000, 2 * (dim // 2) / d_model)
    pe = np.zeros(angles.shape)
    pe[:, 0::2] = np.sin(angles[:, 0::2])
    pe[:, 1::2] = np.cos(angles[:, 1::2])
    return pe.astype(np.float32)


@functools.partial(jax.jit, static_argnums=(3, 4))
def _embed(idx_flat, table, pe2, seq, d):
    ntot = idx_flat.shape[0]
    per_w = ntot // NW
    bpc = 2
    rows = bpc * seq
    orows = rows // 2
    nchunks = per_w // rows
    seg = 80
    nseg = rows // seg
    ngrp = rows // L

    mesh = plsc.VectorSubcoreMesh(core_axis_name="c", subcore_axis_name="s")

    @functools.partial(
        pl.kernel,
        mesh=mesh,
        compiler_params=pltpu.CompilerParams(
            needs_layout_passes=False, use_tc_tiling_on_sc=False
        ),
        out_type=jax.ShapeDtypeStruct((ntot // 2, 2 * d), jnp.float32),
        scratch_types=[
            pltpu.VMEM((rows,), jnp.int32),
            pltpu.VMEM((rows, d), jnp.float32),
            pltpu.VMEM((orows, 2 * d), jnp.float32),
            pltpu.VMEM((seq // 2, 2 * d), jnp.float32),
            pltpu.SemaphoreType.DMA,
        ],
    )
    def body(idx_hbm, table_hbm, pe_hbm, out_hbm, idx_v, gbuf, obuf, pe_v, sem):
        wid = lax.axis_index("s") * NC + lax.axis_index("c")
        base0 = wid * per_w

        pltpu.sync_copy(pe_hbm, pe_v)

        def chunk(ci, _):
            base = base0 + ci * rows
            pltpu.sync_copy(idx_hbm.at[pl.ds(base, rows)], idx_v)
            descs = [
                pltpu.async_copy(
                    table_hbm.at[idx_v.at[pl.ds(g * seg, seg)]],
                    gbuf.at[pl.ds(g * seg, seg)],
                    sem,
                )
                for g in range(nseg)
            ]
            for dsc in descs:
                dsc.wait()

            def scan(g, acc):
                return acc | (idx_v[pl.ds(g * L, L)] == 0)

            m_any = lax.fori_loop(0, ngrp, scan, jnp.zeros((L,), jnp.bool_))
            npad = plsc.all_reduce_population_count(m_any)[0]

            @pl.when(npad > 0)
            def _():
                def fix(g, _):
                    v = idx_v[pl.ds(g * L, L)]
                    m = v == 0
                    row16 = g * L + lax.broadcasted_iota(jnp.int32, (L,), 0)
                    zeros = jnp.zeros((L,), jnp.float32)
                    for j in range(d):
                        col16 = jnp.full((L,), j, jnp.int32)
                        plsc.store_scatter(gbuf, [row16, col16], zeros, mask=m)
                    return 0

                lax.fori_loop(0, ngrp, fix, 0)

            def add_pe(r2, _):
                for h in range(2):
                    for q in range(d // L):
                        c = h * d + q * L
                        obuf[r2, pl.ds(c, L)] = (
                            gbuf[2 * r2 + h, pl.ds(q * L, L)]
                            + pe_v[r2 % (seq // 2), pl.ds(c, L)]
                        )
                return 0

            lax.fori_loop(0, orows, add_pe, 0)

            pltpu.sync_copy(obuf, out_hbm.at[pl.ds(base // 2, orows)])
            return 0

        lax.fori_loop(0, nchunks, chunk, 0)

    return body(idx_flat, table, pe2)


def kernel(input, table):
    b, s = input.shape
    v, d = table.shape
    idx_flat = input.reshape(-1).astype(jnp.int32)
    pe2 = jnp.asarray(_positional_encoding(s, d)).reshape(s // 2, 2 * d)
    out2 = _embed(idx_flat, table, pe2, s, d)
    return out2.reshape(b, s, d)


# speedup vs baseline: 1.4566x; 1.4566x over previous
import functools

import numpy as np
import jax
import jax.numpy as jnp
from jax import lax
from jax.experimental import pallas as pl
from jax.experimental.pallas import tpu as pltpu
from jax.experimental.pallas import tpu_sc as plsc

L = 16
NC, NS = 2, 16
NW = NC * NS


def _positional_encoding(seq_len, d_model):
    pos = np.arange(seq_len)[:, np.newaxis]
    dim = np.arange(d_model)[np.newaxis, :]
    angles = pos / np.power(10000, 2 * (dim // 2) / d_model)
    pe = np.zeros(angles.shape)
    pe[:, 0::2] = np.sin(angles[:, 0::2])
    pe[:, 1::2] = np.cos(angles[:, 1::2])
    return pe.astype(np.float32)


@functools.partial(jax.jit, static_argnums=(3, 4))
def _embed(idx_flat, table, pe2, seq, d):
    ntot = idx_flat.shape[0]
    per_w = ntot // NW
    bpc = 2
    rows = bpc * seq
    nchunks = per_w // rows
    seg = 80
    nseg = rows // seg
    ngrp = rows // L

    mesh = plsc.VectorSubcoreMesh(core_axis_name="c", subcore_axis_name="s")

    @functools.partial(
        pl.kernel,
        mesh=mesh,
        compiler_params=pltpu.CompilerParams(
            needs_layout_passes=False, use_tc_tiling_on_sc=False
        ),
        out_type=jax.ShapeDtypeStruct((ntot, d), jnp.float32),
        scratch_types=[
            pltpu.VMEM((rows,), jnp.int32),
            pltpu.VMEM((rows, d), jnp.float32),
            pltpu.VMEM((rows, d), jnp.float32),
            pltpu.SemaphoreType.DMA,
        ],
    )
    def body(idx_hbm, table_hbm, pe_hbm, out_hbm, idx_v, gbuf, pe_v, sem):
        wid = lax.axis_index("s") * NC + lax.axis_index("c")
        base0 = wid * per_w

        pltpu.sync_copy(pe_hbm, pe_v)

        def chunk(ci, _):
            base = base0 + ci * rows
            pltpu.sync_copy(idx_hbm.at[pl.ds(base, rows)], idx_v)
            descs = [
                pltpu.async_copy(
                    table_hbm.at[idx_v.at[pl.ds(g * seg, seg)]],
                    gbuf.at[pl.ds(g * seg, seg)],
                    sem,
                )
                for g in range(nseg)
            ]
            for dsc in descs:
                dsc.wait()

            def scan(g, acc):
                return acc | (idx_v[pl.ds(g * L, L)] == 0)

            m_any = lax.fori_loop(0, ngrp, scan, jnp.zeros((L,), jnp.bool_))
            npad = plsc.all_reduce_population_count(m_any)[0]

            @pl.when(npad > 0)
            def _():
                def fix(g, _):
                    v = idx_v[pl.ds(g * L, L)]
                    m = v == 0
                    row16 = g * L + lax.broadcasted_iota(jnp.int32, (L,), 0)
                    zeros = jnp.zeros((L,), jnp.float32)
                    for j in range(d):
                        col16 = jnp.full((L,), j, jnp.int32)
                        plsc.store_scatter(gbuf, [row16, col16], zeros, mask=m)
                    return 0

                lax.fori_loop(0, ngrp, fix, 0)

            def add_pe(r, _):
                for q in range(d // L):
                    gbuf[r, pl.ds(q * L, L)] += pe_v[r, pl.ds(q * L, L)]
                return 0

            lax.fori_loop(0, rows, add_pe, 0)

            pltpu.sync_copy(gbuf, out_hbm.at[pl.ds(base, rows)])
            return 0

        lax.fori_loop(0, nchunks, chunk, 0)

    return body(idx_flat, table, pe2)


def kernel(input, table):
    b, s = input.shape
    v, d = table.shape
    idx_flat = input.reshape(-1).astype(jnp.int32)
    pe2 = jnp.asarray(np.tile(_positional_encoding(s, d), (2, 1)))
    out2 = _embed(idx_flat, table, pe2, s, d)
    return out2.reshape(b, s, d)


# speedup vs baseline: 1.8109x; 1.2433x over previous
import functools

import numpy as np
import jax
import jax.numpy as jnp
from jax import lax
from jax.experimental import pallas as pl
from jax.experimental.pallas import tpu as pltpu
from jax.experimental.pallas import tpu_sc as plsc

L = 16
NC, NS = 2, 16
NW = NC * NS


def _positional_encoding(seq_len, d_model):
    pos = np.arange(seq_len)[:, np.newaxis]
    dim = np.arange(d_model)[np.newaxis, :]
    angles = pos / np.power(10000, 2 * (dim // 2) / d_model)
    pe = np.zeros(angles.shape)
    pe[:, 0::2] = np.sin(angles[:, 0::2])
    pe[:, 1::2] = np.cos(angles[:, 1::2])
    return pe.astype(np.float32)


@functools.partial(jax.jit, static_argnums=(3, 4))
def _embed(idx_flat, table, pe2, seq, d):
    ntot = idx_flat.shape[0]
    per_w = ntot // NW
    bpc = 2
    rows = bpc * seq
    nchunks = per_w // rows
    seg = 80
    nseg = rows // seg
    ngrp = rows // L
    nbuf = 4
    nsteps = nchunks // nbuf

    mesh = plsc.VectorSubcoreMesh(core_axis_name="c", subcore_axis_name="s")

    @functools.partial(
        pl.kernel,
        mesh=mesh,
        compiler_params=pltpu.CompilerParams(
            needs_layout_passes=False, use_tc_tiling_on_sc=False
        ),
        out_type=jax.ShapeDtypeStruct((ntot, d), jnp.float32),
        scratch_types=[
            pltpu.VMEM((nbuf, rows), jnp.int32),
            pltpu.VMEM((nbuf, rows, d), jnp.float32),
            pltpu.VMEM((seq, d), jnp.float32),
            [pltpu.SemaphoreType.DMA] * nbuf,
            [pltpu.SemaphoreType.DMA] * nbuf,
        ],
    )
    def body(idx_hbm, table_hbm, pe_hbm, out_hbm, idx_v, gbuf, pe_v, gsem, osem):
        wid = lax.axis_index("s") * NC + lax.axis_index("c")
        base0 = wid * per_w

        pltpu.sync_copy(pe_hbm, pe_v)

        def gdescs(b):
            return [
                pltpu.make_async_copy(
                    table_hbm.at[idx_v.at[b].at[pl.ds(g * seg, seg)]],
                    gbuf.at[b].at[pl.ds(g * seg, seg)],
                    gsem[b],
                )
                for g in range(nseg)
            ]

        def odesc(ci, b):
            return pltpu.make_async_copy(
                gbuf.at[b], out_hbm.at[pl.ds(base0 + ci * rows, rows)],
                osem[b],
            )

        def load(ci, b):
            pltpu.sync_copy(idx_hbm.at[pl.ds(base0 + ci * rows, rows)],
                            idx_v.at[b])
            for dsc in gdescs(b):
                dsc.start()

        def compute(b):
            def scan(g, acc):
                return acc | (idx_v[b, pl.ds(g * L, L)] == 0)

            m_any = lax.fori_loop(0, ngrp, scan, jnp.zeros((L,), jnp.bool_))
            npad = plsc.all_reduce_population_count(m_any)[0]

            @pl.when(npad > 0)
            def _():
                def fix(g, _):
                    v = idx_v[b, pl.ds(g * L, L)]
                    m = v == 0
                    row16 = g * L + lax.broadcasted_iota(jnp.int32, (L,), 0)
                    zeros = jnp.zeros((L,), jnp.float32)
                    for j in range(d):
                        col16 = jnp.full((L,), j, jnp.int32)
                        plsc.store_scatter(
                            gbuf.at[b], [row16, col16], zeros, mask=m
                        )
                    return 0

                lax.fori_loop(0, ngrp, fix, 0)

            def add_pe(r, _):
                for h in range(bpc):
                    for q in range(d // L):
                        sl = pl.ds(q * L, L)
                        gbuf[b, h * seq + r, sl] += pe_v[r, sl]
                return 0

            lax.fori_loop(0, seq, add_pe, 0)

        load(0, 0)

        def step(k, _):
            for i in range(nbuf):
                ci = k * nbuf + i
                nb = (i + 1) % nbuf

                @pl.when(ci + 1 < nchunks)
                def _():
                    @pl.when(ci >= nbuf - 1)
                    def _():
                        odesc(ci - (nbuf - 1), nb).wait()

                    load(ci + 1, nb)

                for dsc in gdescs(i):
                    dsc.wait()

                compute(i)
                odesc(ci, i).start()
            return 0

        lax.fori_loop(0, nsteps, step, 0)

        for i in range(nbuf):
            odesc(nchunks - nbuf + i, i).wait()

    return body(idx_flat, table, pe2)


def kernel(input, table):
    b, s = input.shape
    v, d = table.shape
    idx_flat = input.reshape(-1).astype(jnp.int32)
    pe2 = jnp.asarray(_positional_encoding(s, d))
    out2 = _embed(idx_flat, table, pe2, s, d)
    return out2.reshape(b, s, d)
